# Initial kernel scaffold; baseline (speedup 1.0000x reference)
import jax
import jax.numpy as jnp
from jax.experimental import pallas as pl


def kernel(x, weight):
    raise NotImplementedError("write your pallas kernel here")


# speedup vs baseline: 3.0574x; 3.0574x over previous
import functools

import jax
import jax.numpy as jnp
from jax.experimental import pallas as pl

TOKENS = 32768
DIM = 768
N_EXPERTS = 8
LANES = 128
BLOCK = 2048

NEG_INF = float("-inf")


def _gate_kernel(x_ref, wt_ref, w_out_ref, idx_out_ref):
    x = x_ref[...]
    wt = wt_ref[...]
    s = jnp.dot(x, wt, preferred_element_type=jnp.float32)
    lane = jax.lax.broadcasted_iota(jnp.int32, s.shape, 1)
    s = jnp.where(lane < N_EXPERTS, s, NEG_INF)
    m = jnp.max(s, axis=1, keepdims=True)
    denom = jnp.sum(jnp.exp(s - m), axis=1, keepdims=True)
    w_out_ref[...] = 1.0 / denom
    idx_out_ref[...] = jnp.argmax(s, axis=1).reshape(-1, 1).astype(jnp.int32)


@jax.jit
def kernel(x, weight):
    wt = jnp.zeros((DIM, LANES), dtype=jnp.float32).at[:, :N_EXPERTS].set(
        weight.T)
    grid = (TOKENS // BLOCK,)
    weights, indices = pl.pallas_call(
        _gate_kernel,
        grid=grid,
        in_specs=[
            pl.BlockSpec((BLOCK, DIM), lambda i: (i, 0)),
            pl.BlockSpec((DIM, LANES), lambda i: (0, 0)),
        ],
        out_specs=[
            pl.BlockSpec((BLOCK, 1), lambda i: (i, 0)),
            pl.BlockSpec((BLOCK, 1), lambda i: (i, 0)),
        ],
        out_shape=[
            jax.ShapeDtypeStruct((TOKENS, 1), jnp.float32),
            jax.ShapeDtypeStruct((TOKENS, 1), jnp.int32),
        ],
    )(x, wt)
    return weights, indices


# speedup vs baseline: 3.1707x; 1.0371x over previous
import functools

import jax
import jax.numpy as jnp
from jax.experimental import pallas as pl

TOKENS = 32768
DIM = 768
N_EXPERTS = 8
LANES = 128
BLOCK = 4096

NEG_INF = float("-inf")


def _gate_kernel(x_ref, wt_ref, w_out_ref, idx_out_ref):
    x = x_ref[...]
    wt = wt_ref[...]
    s = jnp.dot(x, wt, preferred_element_type=jnp.float32)
    lane = jax.lax.broadcasted_iota(jnp.int32, s.shape, 1)
    s = jnp.where(lane < N_EXPERTS, s, NEG_INF)
    m = jnp.max(s, axis=1, keepdims=True)
    denom = jnp.sum(jnp.exp(s - m), axis=1, keepdims=True)
    w_out_ref[...] = 1.0 / denom
    idx_out_ref[...] = jnp.argmax(s, axis=1).reshape(-1, 1).astype(jnp.int32)


@jax.jit
def kernel(x, weight):
    wt = jnp.zeros((DIM, LANES), dtype=jnp.float32).at[:, :N_EXPERTS].set(
        weight.T)
    grid = (TOKENS // BLOCK,)
    weights, indices = pl.pallas_call(
        _gate_kernel,
        grid=grid,
        in_specs=[
            pl.BlockSpec((BLOCK, DIM), lambda i: (i, 0)),
            pl.BlockSpec((DIM, LANES), lambda i: (0, 0)),
        ],
        out_specs=[
            pl.BlockSpec((BLOCK, 1), lambda i: (i, 0)),
            pl.BlockSpec((BLOCK, 1), lambda i: (i, 0)),
        ],
        out_shape=[
            jax.ShapeDtypeStruct((TOKENS, 1), jnp.float32),
            jax.ShapeDtypeStruct((TOKENS, 1), jnp.int32),
        ],
    )(x, wt)
    return weights, indices
